# Initial kernel scaffold; baseline (speedup 1.0000x reference)
import jax
import jax.numpy as jnp
from jax.experimental import pallas as pl


def kernel(memory, last_update, unique_msg, time, W_ih, W_hh, b_ih, b_hh, unique_node_ids):
    raise NotImplementedError("write your pallas kernel here")


# speedup vs baseline: 5.2240x; 5.2240x over previous
import jax
import jax.numpy as jnp
from jax import lax
from jax.experimental import pallas as pl
from jax.experimental.pallas import tpu as pltpu
from jax.experimental.pallas import tpu_sc as plsc
from jax._src.pallas.mpmd import _mpmd_map

NC, NS, L = 2, 16, 16
NW = NC * NS


def _wid():
    return lax.axis_index("s") * NC + lax.axis_index("c")


def _sc_mesh():
    return plsc.VectorSubcoreMesh(core_axis_name="c", subcore_axis_name="s")


def _row_dma_loop(idx_v, n, issue):
    lanes = lax.iota(jnp.int32, L)

    def group(g, c):
        vec = idx_v[pl.ds(g * L, L)]
        for r in range(L):
            issue(g * L + r, jnp.sum(jnp.where(lanes == r, vec, 0)))
        return c

    lax.fori_loop(0, n // L, group, 0)


def _make_sc_gather(M, D, B):
    bpw = B // NW

    def body(ids_hbm, mem_hbm, out_hbm, idx_v, rows_v, sem):
        base = _wid() * bpw
        pltpu.sync_copy(ids_hbm.at[pl.ds(base, bpw)], idx_v)

        _row_dma_loop(idx_v, bpw, lambda k, row: pltpu.async_copy(
            mem_hbm.at[pl.ds(row, 1)], rows_v.at[pl.ds(k, 1)], sem))
        pltpu.make_async_copy(mem_hbm.at[pl.ds(0, bpw)], rows_v, sem).wait()
        pltpu.sync_copy(rows_v, out_hbm.at[pl.ds(base, bpw)])

    return pl.kernel(
        body,
        out_type=jax.ShapeDtypeStruct((B, D), jnp.float32),
        mesh=_sc_mesh(),
        scratch_types=[
            pltpu.VMEM((bpw,), jnp.int32),
            pltpu.VMEM((bpw, D), jnp.float32),
            pltpu.SemaphoreType.DMA,
        ],
        compiler_params=pltpu.CompilerParams(needs_layout_passes=False),
    )


def _make_tc_gru(M, D, B, MSG, nb):
    bm = B // nb

    def body(ids_ref, msgt_ref, h_ref,
             wr_ih, wz_ih, wn_ih, wr_hh, wz_hh, wn_hh,
             br_ref, bz_ref, bin_ref, bhn_ref,
             hnew_ref, jw_ref):
        s = pl.program_id(0)
        msgt = msgt_ref[...].astype(jnp.bfloat16)
        h = h_ref[...]
        hb = h.astype(jnp.bfloat16)

        def mmt(wref):
            return lax.dot_general(
                msgt, wref[...],
                dimension_numbers=(((0,), (1,)), ((), ())),
                preferred_element_type=jnp.float32,
            )

        def mmh(wref):
            return lax.dot_general(
                hb, wref[...],
                dimension_numbers=(((1,), (1,)), ((), ())),
                preferred_element_type=jnp.float32,
            )

        r = jax.nn.sigmoid(mmt(wr_ih) + mmh(wr_hh) + br_ref[...])
        z = jax.nn.sigmoid(mmt(wz_ih) + mmh(wz_hh) + bz_ref[...])
        n = jnp.tanh(mmt(wn_ih) + bin_ref[...]
                     + r * (mmh(wn_hh) + bhn_ref[...]))
        hnew_ref[...] = (1.0 - z) * n + z * h

        ids_all = ids_ref[...]
        idblk = ids_ref[pl.ds(s * bm, bm)]
        eq = idblk[:, None] == ids_all[None, :]
        jall = lax.broadcasted_iota(jnp.int32, (bm, B), 1).astype(jnp.float32)
        jwf = jnp.max(jnp.where(eq, jall, -1.0), axis=1)
        jw_ref[...] = jwf.astype(jnp.int32)

    return pl.pallas_call(
        body,
        grid=(nb,),
        in_specs=[
            pl.BlockSpec((B,), lambda s: (0,)),
            pl.BlockSpec((MSG, bm), lambda s: (0, s)),
            pl.BlockSpec((bm, D), lambda s: (s, 0)),
            pl.BlockSpec((D, MSG), lambda s: (0, 0)),
            pl.BlockSpec((D, MSG), lambda s: (0, 0)),
            pl.BlockSpec((D, MSG), lambda s: (0, 0)),
            pl.BlockSpec((D, D), lambda s: (0, 0)),
            pl.BlockSpec((D, D), lambda s: (0, 0)),
            pl.BlockSpec((D, D), lambda s: (0, 0)),
            pl.BlockSpec((1, D), lambda s: (0, 0)),
            pl.BlockSpec((1, D), lambda s: (0, 0)),
            pl.BlockSpec((1, D), lambda s: (0, 0)),
            pl.BlockSpec((1, D), lambda s: (0, 0)),
        ],
        out_specs=[
            pl.BlockSpec((bm, D), lambda s: (s, 0)),
            pl.BlockSpec((bm,), lambda s: (s,)),
        ],
        out_shape=[
            jax.ShapeDtypeStruct((B, D), jnp.float32),
            jax.ShapeDtypeStruct((B,), jnp.int32),
        ],
        compiler_params=pltpu.CompilerParams(
            dimension_semantics=("arbitrary",),
            vmem_limit_bytes=100 * 1024 * 1024,
        ),
    )


def _make_sc_scatter(M, D, B):
    bpw = B // NW

    def body(base_ref, hnew_hbm, jw_hbm, ids_hbm, out_hbm,
             idx_v, jw_v, rows_v, sem):
        del base_ref
        base = _wid() * bpw
        pltpu.sync_copy(ids_hbm.at[pl.ds(base, bpw)], idx_v)
        pltpu.sync_copy(jw_hbm.at[pl.ds(base, bpw)], jw_v)

        _row_dma_loop(jw_v, bpw, lambda k, row: pltpu.async_copy(
            hnew_hbm.at[pl.ds(row, 1)], rows_v.at[pl.ds(k, 1)], sem))
        pltpu.make_async_copy(hnew_hbm.at[pl.ds(0, bpw)], rows_v, sem).wait()

        _row_dma_loop(idx_v, bpw, lambda k, row: pltpu.async_copy(
            rows_v.at[pl.ds(k, 1)], out_hbm.at[pl.ds(row, 1)], sem))
        pltpu.make_async_copy(rows_v, out_hbm.at[pl.ds(0, bpw)], sem).wait()

    kern = _mpmd_map(
        [(_sc_mesh(), body)],
        jax.ShapeDtypeStruct((M, D), jnp.float32),
        input_output_aliases={0: 0},
        compiler_params=pltpu.CompilerParams(needs_layout_passes=False),
        scratch_types=[
            pltpu.VMEM((bpw,), jnp.int32),
            pltpu.VMEM((bpw,), jnp.int32),
            pltpu.VMEM((bpw, D), jnp.float32),
            pltpu.SemaphoreType.DMA,
        ],
    )
    return kern


def _make_sc_last(M, B):
    seg = ((M // NW) + 7) // 8 * 8
    last_seg = M - seg * (NW - 1)
    assert last_seg > 0 and last_seg % 8 == 0
    iters = B // L

    def body(last_hbm, ids_hbm, jw_hbm, time_hbm, out_hbm,
             seg_v, ids_v, jw_v, time_v):
        w = _wid()
        lo = w * seg
        size = jnp.where(w == NW - 1, last_seg, seg)

        @pl.when(w < NW - 1)
        def _():
            pltpu.sync_copy(last_hbm.at[pl.ds(lo, seg)], seg_v)

        @pl.when(w == NW - 1)
        def _():
            pltpu.sync_copy(last_hbm.at[pl.ds((NW - 1) * seg, last_seg)],
                            seg_v.at[pl.ds(0, last_seg)])

        pltpu.sync_copy(ids_hbm, ids_v)
        pltpu.sync_copy(jw_hbm, jw_v)
        pltpu.sync_copy(time_hbm, time_v)

        def step(k, carry):
            idv = ids_v[pl.ds(k * L, L)]
            jwv = jw_v[pl.ds(k * L, L)]
            tv = plsc.load_gather(time_v, [jwv])
            m = (idv >= lo) & (idv < lo + size)
            plsc.store_scatter(seg_v, [idv - lo], tv, mask=m)
            return carry

        lax.fori_loop(0, iters, step, 0)

        @pl.when(w < NW - 1)
        def _():
            pltpu.sync_copy(seg_v, out_hbm.at[pl.ds(lo, seg)])

        @pl.when(w == NW - 1)
        def _():
            pltpu.sync_copy(seg_v.at[pl.ds(0, last_seg)],
                            out_hbm.at[pl.ds((NW - 1) * seg, last_seg)])

    return pl.kernel(
        body,
        out_type=jax.ShapeDtypeStruct((M,), jnp.float32),
        mesh=_sc_mesh(),
        scratch_types=[
            pltpu.VMEM((seg,), jnp.float32),
            pltpu.VMEM((B,), jnp.int32),
            pltpu.VMEM((B,), jnp.int32),
            pltpu.VMEM((B,), jnp.float32),
        ],
        compiler_params=pltpu.CompilerParams(needs_layout_passes=False),
    )


def kernel(memory, last_update, unique_msg, time, W_ih, W_hh, b_ih, b_hh,
           unique_node_ids):
    M, D = memory.shape
    B, MSG = unique_msg.shape
    nb = 16

    br = (b_ih[0:D] + b_hh[0:D]).reshape(1, D)
    bz = (b_ih[D:2 * D] + b_hh[D:2 * D]).reshape(1, D)
    bin_ = b_ih[2 * D:3 * D].reshape(1, D)
    bhn = b_hh[2 * D:3 * D].reshape(1, D)

    h = _make_sc_gather(M, D, B)(unique_node_ids, memory)

    bf = jnp.bfloat16
    h_new, jw = _make_tc_gru(M, D, B, MSG, nb)(
        unique_node_ids, unique_msg.T, h,
        W_ih[0:D].astype(bf), W_ih[D:2 * D].astype(bf),
        W_ih[2 * D:3 * D].astype(bf),
        W_hh[0:D].astype(bf), W_hh[D:2 * D].astype(bf),
        W_hh[2 * D:3 * D].astype(bf),
        br, bz, bin_, bhn,
    )

    updated_memory = _make_sc_scatter(M, D, B)(
        memory, h_new, jw, unique_node_ids)

    updated_last = _make_sc_last(M, B)(
        last_update, unique_node_ids, jw, time)

    return (updated_memory, updated_last)
